# baseline (speedup 1.0000x reference)
import jax
import jax.numpy as jnp
from jax import lax
from jax.experimental import pallas as pl
from jax.experimental.pallas import tpu as pltpu
from jax.experimental.pallas import tpu_sc as plsc

SEQ = 200
BATCH = 1024
DM = 64
NV = 4
LANES = 16
NC, NS = 2, 16
NW = NC * NS
ROWS_PER_W = BATCH // NW
CH = 40
NCHUNK = SEQ // CH
WTOK = ROWS_PER_W * SEQ
IPAD = 208
TOK = BATCH * SEQ
IDXB = NV * CH


def _body(xTc, W0r, W1r, W2r, W3r, Wp45, out,
          xidxA, xidxB, kA, kB, pA, pB, pidx_s, pidx_f, w6v, wrc,
          gbA, gbB, gb45, semA, semB, isemA, isemB, wsem):
    wid = lax.axis_index("s") * NC + lax.axis_index("c")
    tbase = wid * WTOK
    gbase = wid * ROWS_PER_W * NCHUNK
    tables = (W0r, W1r, W2r, W3r)

    iota = lax.iota(jnp.int32, LANES)
    for i in range(IPAD // LANES):
        s = iota + (i * LANES)
        pidx_s[pl.ds(i * LANES, LANES)] = jnp.minimum(s, SEQ - 1)
        pidx_f[pl.ds(i * LANES, LANES)] = SEQ + jnp.clip(s - 149, 0, 50)

    pltpu.sync_copy(Wp45.at[pl.ds(248, 8)], w6v)
    w6lo = [w6v[3, pl.ds(c * LANES, LANES)] for c in range(DM // LANES)]
    w6hi = [w6v[3, pl.ds(DM + c * LANES, LANES)]
            for c in range(DM // LANES)]
    for c in range(NCHUNK):
        dst = gb45.at[c]
        pltpu.async_copy(Wp45.at[pidx_s.at[pl.ds(c * CH, CH)]],
                         dst, semA).wait()
        pltpu.async_copy(Wp45.at[pidx_f.at[pl.ds(c * CH, CH)]],
                         dst, semA, add=True).wait()

    def fire_idx(i, cpos, xi, isem):
        gid = gbase + i * NCHUNK + cpos
        pltpu.async_copy(xTc.at[pl.ds(gid * IDXB, IDXB)], xi, isem)

    def drain_idx(xi, isem):
        pltpu.make_async_copy(xTc.at[pl.ds(0, IDXB)], xi, isem).wait()

    def compute_kp(xi, k, p):
        for g in range(IDXB // LANES):
            xv = xi[pl.ds(g * LANES, LANES)]
            k[pl.ds(g * LANES, LANES)] = lax.shift_right_logical(xv, 1)
            p[pl.ds(g * LANES, LANES)] = lax.bitwise_and(xv, 1)

    def fire_g(k, gb, sem):
        for v in range(NV):
            pltpu.async_copy(tables[v].at[k.at[pl.ds(v * CH, CH)]],
                             gb.at[v], sem)

    def drain_g(gb, sem):
        for v in range(NV):
            pltpu.make_async_copy(W0r.at[pl.ds(0, CH)], gb.at[v],
                                  sem).wait()

    def drain_w():
        pltpu.make_async_copy(wrc, out.at[pl.ds(0, CH)], wsem).wait()

    def unpack_data(gb, p):
        def step(t, carry):
            for v in range(NV):
                off = p[pl.ds(v * CH + t, LANES)][0] * DM
                for c in range(DM // LANES):
                    wrc[t, v, pl.ds(c * LANES, LANES)] = (
                        gb[v, t, pl.ds(off + c * LANES, LANES)])
            return carry
        lax.fori_loop(0, CH, step, 0)

    def write(i, cpos):
        pltpu.async_copy(wrc, out.at[pl.ds(tbase + i * SEQ + cpos * CH, CH)],
                         wsem)

    pltpu.async_copy(wrc, out.at[pl.ds(tbase, CH)], wsem)

    for cpos in range(NCHUNK):
        drain_w()
        lo_all = (cpos + 1) * CH <= SEQ - 50
        hi_all = cpos * CH >= SEQ - 50

        def pos_step(t, carry):
            for c in range(2 * DM // LANES):
                wrc[t, 4 + c // 4, pl.ds((c % 4) * LANES, LANES)] = (
                    gb45[cpos, t, pl.ds(c * LANES, LANES)])
            for c in range(DM // LANES):
                if lo_all:
                    vec = w6lo[c]
                elif hi_all:
                    vec = w6hi[c]
                else:
                    vec = jnp.where(cpos * CH + t >= SEQ - 50,
                                    w6hi[c], w6lo[c])
                wrc[t, 6, pl.ds(c * LANES, LANES)] = vec
            return carry
        lax.fori_loop(0, CH, pos_step, 0)

        pltpu.async_copy(wrc, out.at[pl.ds(tbase + cpos * CH, CH)], wsem)
        fire_idx(jnp.int32(0), cpos, xidxA, isemA)
        fire_idx(jnp.int32(1), cpos, xidxB, isemB)
        drain_idx(xidxA, isemA)
        compute_kp(xidxA, kA, pA)
        fire_g(kA, gbA, semA)
        drain_idx(xidxB, isemB)
        compute_kp(xidxB, kB, pB)
        fire_g(kB, gbB, semB)

        def seg_body(j, carry):
            i = 2 * j
            drain_g(gbA, semA)
            fire_idx(jnp.minimum(i + 2, ROWS_PER_W - 1), cpos, xidxA, isemA)
            drain_w()
            unpack_data(gbA, pA)
            write(i, cpos)
            drain_idx(xidxA, isemA)
            compute_kp(xidxA, kA, pA)
            fire_g(kA, gbA, semA)

            drain_g(gbB, semB)
            fire_idx(jnp.minimum(i + 3, ROWS_PER_W - 1), cpos, xidxB, isemB)
            drain_w()
            unpack_data(gbB, pB)
            write(i + 1, cpos)
            drain_idx(xidxB, isemB)
            compute_kp(xidxB, kB, pB)
            fire_g(kB, gbB, semB)
            return carry
        lax.fori_loop(0, ROWS_PER_W // 2, seg_body, 0)

        drain_g(gbA, semA)
        drain_g(gbB, semB)

    drain_w()


def kernel(x, W0, W1, W2, W3, W4, W5, W6):
    xTc = jnp.transpose(
        x.astype(jnp.int32).reshape(BATCH, NCHUNK, CH, NV),
        (0, 1, 3, 2)).reshape(-1)
    Wr = [w.reshape(50000, 2 * DM) for w in (W0, W1, W2, W3)]
    Wp45 = jnp.concatenate([
        jnp.concatenate([W4, jnp.zeros_like(W4)], axis=1),
        jnp.concatenate([jnp.zeros_like(W5), W5], axis=1),
        W6.reshape(1, 2 * DM),
        jnp.zeros((4, 2 * DM), jnp.float32),
    ], axis=0)

    mesh = plsc.VectorSubcoreMesh(core_axis_name="c", subcore_axis_name="s")
    f = pl.kernel(
        _body,
        out_type=jax.ShapeDtypeStruct((TOK, 7, DM), jnp.float32),
        mesh=mesh,
        scratch_types=[
            pltpu.VMEM((IDXB,), jnp.int32),
            pltpu.VMEM((IDXB,), jnp.int32),
            pltpu.VMEM((IDXB,), jnp.int32),
            pltpu.VMEM((IDXB,), jnp.int32),
            pltpu.VMEM((IDXB + LANES,), jnp.int32),
            pltpu.VMEM((IDXB + LANES,), jnp.int32),
            pltpu.VMEM((IPAD,), jnp.int32),
            pltpu.VMEM((IPAD,), jnp.int32),
            pltpu.VMEM((8, 2 * DM), jnp.float32),
            pltpu.VMEM((CH, 7, DM), jnp.float32),
            pltpu.VMEM((NV, CH, 2 * DM), jnp.float32),
            pltpu.VMEM((NV, CH, 2 * DM), jnp.float32),
            pltpu.VMEM((NCHUNK, CH, 2 * DM), jnp.float32),
            pltpu.SemaphoreType.DMA,
            pltpu.SemaphoreType.DMA,
            pltpu.SemaphoreType.DMA,
            pltpu.SemaphoreType.DMA,
            pltpu.SemaphoreType.DMA,
        ],
    )
    out = f(xTc, *Wr, Wp45)
    return out.reshape(BATCH, SEQ, 7, DM)


# speedup vs baseline: 1.2240x; 1.2240x over previous
import jax
import jax.numpy as jnp
from jax import lax
from jax.experimental import pallas as pl
from jax.experimental.pallas import tpu as pltpu
from jax.experimental.pallas import tpu_sc as plsc

SEQ = 200
BATCH = 1024
DM = 64
NV = 4
LANES = 16
NC, NS = 2, 16
NW = NC * NS
ROWS_PER_W = BATCH // NW
CH = 40
NCHUNK = SEQ // CH
WTOK = ROWS_PER_W * SEQ
NPAD = NCHUNK * CH + 0
IPAD = 208
TOK = BATCH * SEQ


def _body(xTc, Wd0, Wd1, Wd2, Wd3, Wp45, out,
          xidxA, xidxB, pidx_s, pidx_f, w6v, wrc, gbA, gbB, gb45,
          semA, semB, wsem):
    wid = lax.axis_index("s") * NC + lax.axis_index("c")
    tbase = wid * WTOK
    gbase = wid * ROWS_PER_W * NCHUNK
    tables = (Wd0, Wd1, Wd2, Wd3)

    iota = lax.iota(jnp.int32, LANES)
    for i in range(IPAD // LANES):
        s = iota + (i * LANES)
        pidx_s[pl.ds(i * LANES, LANES)] = jnp.minimum(s, SEQ - 1)
        pidx_f[pl.ds(i * LANES, LANES)] = SEQ + jnp.clip(s - 149, 0, 50)

    pltpu.sync_copy(Wp45.at[pl.ds(248, 8)], w6v)
    w6lo = [w6v[3, pl.ds(c * LANES, LANES)] for c in range(DM // LANES)]
    w6hi = [w6v[3, pl.ds(DM + c * LANES, LANES)]
            for c in range(DM // LANES)]
    for c in range(NCHUNK):
        dst = gb45.at[c]
        pltpu.async_copy(Wp45.at[pidx_s.at[pl.ds(c * CH, CH)]],
                         dst, semA).wait()
        pltpu.async_copy(Wp45.at[pidx_f.at[pl.ds(c * CH, CH)]],
                         dst, semA, add=True).wait()

    def fire(i, cpos, xi, gb, sem):
        gid = gbase + i * NCHUNK + cpos
        pltpu.sync_copy(xTc.at[pl.ds(gid * (NV * CH), NV * CH)], xi)
        for v in range(NV):
            pltpu.async_copy(tables[v].at[xi.at[pl.ds(v * CH, CH)]],
                             gb.at[v], sem)

    def drain_g(gb, sem):
        for v in range(NV):
            pltpu.make_async_copy(Wd0.at[pl.ds(0, CH)], gb.at[v], sem).wait()

    def drain_w():
        pltpu.make_async_copy(wrc, out.at[pl.ds(0, CH)], wsem).wait()

    def unpack_data(gb):
        def step(t, carry):
            for v in range(NV):
                for c in range(DM // LANES):
                    wrc[t, v, pl.ds(c * LANES, LANES)] = (
                        gb[v, t, pl.ds(c * LANES, LANES)])
            return carry
        lax.fori_loop(0, CH, step, 0)

    def write(i, cpos):
        pltpu.async_copy(wrc, out.at[pl.ds(tbase + i * SEQ + cpos * CH, CH)],
                         wsem)

    pltpu.async_copy(wrc, out.at[pl.ds(tbase, CH)], wsem)

    for cpos in range(NCHUNK):
        drain_w()
        lo_all = (cpos + 1) * CH <= SEQ - 50
        hi_all = cpos * CH >= SEQ - 50

        def pos_step(t, carry):
            for c in range(2 * DM // LANES):
                wrc[t, 4 + c // 4, pl.ds((c % 4) * LANES, LANES)] = (
                    gb45[cpos, t, pl.ds(c * LANES, LANES)])
            for c in range(DM // LANES):
                if lo_all:
                    vec = w6lo[c]
                elif hi_all:
                    vec = w6hi[c]
                else:
                    vec = jnp.where(cpos * CH + t >= SEQ - 50,
                                    w6hi[c], w6lo[c])
                wrc[t, 6, pl.ds(c * LANES, LANES)] = vec
            return carry
        lax.fori_loop(0, CH, pos_step, 0)

        pltpu.async_copy(wrc, out.at[pl.ds(tbase + cpos * CH, CH)], wsem)
        fire(jnp.int32(0), cpos, xidxA, gbA, semA)
        fire(jnp.int32(1), cpos, xidxB, gbB, semB)

        def seg_body(j, carry):
            i = 2 * j
            drain_g(gbA, semA)
            drain_w()
            unpack_data(gbA)
            write(i, cpos)
            fire(jnp.minimum(i + 2, ROWS_PER_W - 1), cpos, xidxA, gbA, semA)
            drain_g(gbB, semB)
            drain_w()
            unpack_data(gbB)
            write(i + 1, cpos)
            fire(jnp.minimum(i + 3, ROWS_PER_W - 1), cpos, xidxB, gbB, semB)
            return carry
        lax.fori_loop(0, ROWS_PER_W // 2, seg_body, 0)

        drain_g(gbA, semA)
        drain_g(gbB, semB)

    drain_w()


def kernel(x, W0, W1, W2, W3, W4, W5, W6):
    xTc = jnp.transpose(
        x.astype(jnp.int32).reshape(BATCH, NCHUNK, CH, NV),
        (0, 1, 3, 2)).reshape(-1)
    Wd = [jnp.concatenate([w, w], axis=1) for w in (W0, W1, W2, W3)]
    Wp45 = jnp.concatenate([
        jnp.concatenate([W4, jnp.zeros_like(W4)], axis=1),
        jnp.concatenate([jnp.zeros_like(W5), W5], axis=1),
        W6.reshape(1, 2 * DM),
        jnp.zeros((4, 2 * DM), jnp.float32),
    ], axis=0)

    mesh = plsc.VectorSubcoreMesh(core_axis_name="c", subcore_axis_name="s")
    f = pl.kernel(
        _body,
        out_type=jax.ShapeDtypeStruct((TOK, 7, DM), jnp.float32),
        mesh=mesh,
        scratch_types=[
            pltpu.VMEM((NV * CH,), jnp.int32),
            pltpu.VMEM((NV * CH,), jnp.int32),
            pltpu.VMEM((IPAD,), jnp.int32),
            pltpu.VMEM((IPAD,), jnp.int32),
            pltpu.VMEM((8, 2 * DM), jnp.float32),
            pltpu.VMEM((CH, 7, DM), jnp.float32),
            pltpu.VMEM((NV, CH, 2 * DM), jnp.float32),
            pltpu.VMEM((NV, CH, 2 * DM), jnp.float32),
            pltpu.VMEM((NCHUNK, CH, 2 * DM), jnp.float32),
            pltpu.SemaphoreType.DMA,
            pltpu.SemaphoreType.DMA,
            pltpu.SemaphoreType.DMA,
        ],
    )
    out = f(xTc, *Wd, Wp45)
    return out.reshape(BATCH, SEQ, 7, DM)


# speedup vs baseline: 1.2318x; 1.0064x over previous
import jax
import jax.numpy as jnp
from jax import lax
from jax.experimental import pallas as pl
from jax.experimental.pallas import tpu as pltpu
from jax.experimental.pallas import tpu_sc as plsc

SEQ = 200
BATCH = 1024
DM = 64
NV = 4
LANES = 16
NC, NS = 2, 16
NW = NC * NS
ROWS_PER_W = BATCH // NW
CH = 40
NCHUNK = SEQ // CH
WTOK = ROWS_PER_W * SEQ
NPAD = NCHUNK * CH + 0
IPAD = 208
TOK = BATCH * SEQ


def _body(xTc, Wd0, Wd1, Wd2, Wd3, WL4, WR5, W6f, out,
          xidxA, xidxB, pidx_s, pidx_f, w6v, wrc, gbA, gbB, gb45,
          semA, semB, wsem):
    wid = lax.axis_index("s") * NC + lax.axis_index("c")
    tbase = wid * WTOK
    gbase = wid * ROWS_PER_W * NCHUNK
    tables = (Wd0, Wd1, Wd2, Wd3)

    iota = lax.iota(jnp.int32, LANES)
    for i in range(IPAD // LANES):
        s = iota + (i * LANES)
        pidx_s[pl.ds(i * LANES, LANES)] = jnp.minimum(s, SEQ - 1)
        pidx_f[pl.ds(i * LANES, LANES)] = jnp.clip(s - 149, 0, 50)

    pltpu.sync_copy(W6f, w6v)
    w6lo = [w6v[pl.ds(c * LANES, LANES)] for c in range(DM // LANES)]
    w6hi = [w6v[pl.ds(DM + c * LANES, LANES)] for c in range(DM // LANES)]
    for c in range(NCHUNK):
        dst = gb45.at[c]
        pltpu.async_copy(WL4.at[pidx_s.at[pl.ds(c * CH, CH)]],
                         dst, semA).wait()
        pltpu.async_copy(WR5.at[pidx_f.at[pl.ds(c * CH, CH)]],
                         dst, semA, add=True).wait()

    def fire(i, cpos, xi, gb, sem):
        gid = gbase + i * NCHUNK + cpos
        pltpu.sync_copy(xTc.at[pl.ds(gid * (NV * CH), NV * CH)], xi)
        for v in range(NV):
            pltpu.async_copy(tables[v].at[xi.at[pl.ds(v * CH, CH)]],
                             gb.at[v], sem)

    def drain_g(gb, sem):
        for v in range(NV):
            pltpu.make_async_copy(Wd0.at[pl.ds(0, CH)], gb.at[v], sem).wait()

    def drain_w():
        pltpu.make_async_copy(wrc, out.at[pl.ds(0, CH)], wsem).wait()

    def unpack_data(gb):
        def step(t, carry):
            for v in range(NV):
                for c in range(DM // LANES):
                    wrc[t, v, pl.ds(c * LANES, LANES)] = (
                        gb[v, t, pl.ds(c * LANES, LANES)])
            return carry
        lax.fori_loop(0, CH, step, 0)

    def write(i, cpos):
        pltpu.async_copy(wrc, out.at[pl.ds(tbase + i * SEQ + cpos * CH, CH)],
                         wsem)

    pltpu.async_copy(wrc, out.at[pl.ds(tbase, CH)], wsem)

    for cpos in range(NCHUNK):
        drain_w()
        lo_all = (cpos + 1) * CH <= SEQ - 50
        hi_all = cpos * CH >= SEQ - 50

        def pos_step(t, carry):
            for c in range(2 * DM // LANES):
                wrc[t, 4 + c // 4, pl.ds((c % 4) * LANES, LANES)] = (
                    gb45[cpos, t, pl.ds(c * LANES, LANES)])
            for c in range(DM // LANES):
                if lo_all:
                    vec = w6lo[c]
                elif hi_all:
                    vec = w6hi[c]
                else:
                    vec = jnp.where(cpos * CH + t >= SEQ - 50,
                                    w6hi[c], w6lo[c])
                wrc[t, 6, pl.ds(c * LANES, LANES)] = vec
            return carry
        lax.fori_loop(0, CH, pos_step, 0)

        pltpu.async_copy(wrc, out.at[pl.ds(tbase + cpos * CH, CH)], wsem)
        fire(jnp.int32(0), cpos, xidxA, gbA, semA)
        fire(jnp.int32(1), cpos, xidxB, gbB, semB)

        def seg_body(j, carry):
            i = 2 * j
            drain_g(gbA, semA)
            drain_w()
            unpack_data(gbA)
            write(i, cpos)
            fire(jnp.minimum(i + 2, ROWS_PER_W - 1), cpos, xidxA, gbA, semA)
            drain_g(gbB, semB)
            drain_w()
            unpack_data(gbB)
            write(i + 1, cpos)
            fire(jnp.minimum(i + 3, ROWS_PER_W - 1), cpos, xidxB, gbB, semB)
            return carry
        lax.fori_loop(0, ROWS_PER_W // 2, seg_body, 0)

        drain_g(gbA, semA)
        drain_g(gbB, semB)

    drain_w()


def _pair(w, side):
    z = jnp.zeros_like(w)
    cols = (w, z) if side == 0 else (z, w)
    return jnp.concatenate(cols, axis=1)


def kernel(x, W0, W1, W2, W3, W4, W5, W6):
    xTc = jnp.transpose(
        x.astype(jnp.int32).reshape(BATCH, NCHUNK, CH, NV),
        (0, 1, 3, 2)).reshape(-1)
    Wd = [jnp.concatenate([w, w], axis=1) for w in (W0, W1, W2, W3)]
    WL4, WR5 = _pair(W4, 0), _pair(W5, 1)
    W6f = W6.reshape(2 * DM)

    mesh = plsc.VectorSubcoreMesh(core_axis_name="c", subcore_axis_name="s")
    f = pl.kernel(
        _body,
        out_type=jax.ShapeDtypeStruct((TOK, 7, DM), jnp.float32),
        mesh=mesh,
        scratch_types=[
            pltpu.VMEM((NV * CH,), jnp.int32),
            pltpu.VMEM((NV * CH,), jnp.int32),
            pltpu.VMEM((IPAD,), jnp.int32),
            pltpu.VMEM((IPAD,), jnp.int32),
            pltpu.VMEM((2 * DM,), jnp.float32),
            pltpu.VMEM((CH, 7, DM), jnp.float32),
            pltpu.VMEM((NV, CH, 2 * DM), jnp.float32),
            pltpu.VMEM((NV, CH, 2 * DM), jnp.float32),
            pltpu.VMEM((NCHUNK, CH, 2 * DM), jnp.float32),
            pltpu.SemaphoreType.DMA,
            pltpu.SemaphoreType.DMA,
            pltpu.SemaphoreType.DMA,
        ],
    )
    out = f(xTc, *Wd, WL4, WR5, W6f)
    return out.reshape(BATCH, SEQ, 7, DM)


# speedup vs baseline: 1.3034x; 1.0581x over previous
import jax
import jax.numpy as jnp
from jax import lax
from jax.experimental import pallas as pl
from jax.experimental.pallas import tpu as pltpu
from jax.experimental.pallas import tpu_sc as plsc

SEQ = 200
BATCH = 1024
DM = 64
NV = 4
LANES = 16
NC, NS = 2, 16
NW = NC * NS
ROWS_PER_W = BATCH // NW
CH = 40
NCHUNK = SEQ // CH
WTOK = ROWS_PER_W * SEQ
NPAD = NCHUNK * CH + 0
IPAD = 208
TOK = BATCH * SEQ
CHH = CH // 2


def _body(xTc, Wd0, Wd1, Wd2, Wd3, WL4, WR5, W6f, out,
          xidxA, xidxB, pidx_s, pidx_f, w6v, wrc, gbA, gbB, gb45,
          semA, semB, wsem):
    wid = lax.axis_index("s") * NC + lax.axis_index("c")
    tbase = wid * WTOK
    gbase = wid * ROWS_PER_W * NCHUNK
    tables = (Wd0, Wd1, Wd2, Wd3)

    iota = lax.iota(jnp.int32, LANES)
    for i in range(IPAD // LANES):
        s = iota + (i * LANES)
        pidx_s[pl.ds(i * LANES, LANES)] = jnp.minimum(s, SEQ - 1)
        pidx_f[pl.ds(i * LANES, LANES)] = jnp.clip(s - 149, 0, 50)

    pltpu.sync_copy(W6f, w6v)
    w6lo = [w6v[pl.ds(c * LANES, LANES)] for c in range(DM // LANES)]
    w6hi = [w6v[pl.ds(DM + c * LANES, LANES)] for c in range(DM // LANES)]
    for c in range(NCHUNK):
        dst = gb45.at[c]
        pltpu.async_copy(WL4.at[pidx_s.at[pl.ds(c * CH, CH)]],
                         dst, semA).wait()
        pltpu.async_copy(WR5.at[pidx_f.at[pl.ds(c * CH, CH)]],
                         dst, semA, add=True).wait()

    def fire(i, cpos, xi, gb, sem):
        gid = gbase + i * NCHUNK + cpos
        pltpu.sync_copy(xTc.at[pl.ds(gid * (NV * CH), NV * CH)], xi)
        for v in range(NV):
            pltpu.async_copy(tables[v].at[xi.at[pl.ds(v * CH, CH)]],
                             gb.at[v], sem)

    def drain_g(gb, sem):
        for v in range(NV):
            pltpu.make_async_copy(Wd0.at[pl.ds(0, CH)], gb.at[v], sem).wait()

    def drain_w():
        pltpu.make_async_copy(wrc.at[0], out.at[pl.ds(0, CHH)], wsem).wait()

    def unpack_half(gb, h):
        def step(t, carry):
            for v in range(NV):
                for c in range(DM // LANES):
                    wrc[h, t, v, pl.ds(c * LANES, LANES)] = (
                        gb[v, h * CHH + t, pl.ds(c * LANES, LANES)])
            return carry
        lax.fori_loop(0, CHH, step, 0)

    def write(i, cpos, h):
        pltpu.async_copy(
            wrc.at[h],
            out.at[pl.ds(tbase + i * SEQ + cpos * CH + h * CHH, CHH)], wsem)

    pltpu.async_copy(wrc.at[0], out.at[pl.ds(tbase, CHH)], wsem)
    pltpu.async_copy(wrc.at[1], out.at[pl.ds(tbase + CHH, CHH)], wsem)

    for cpos in range(NCHUNK):
        drain_w()
        drain_w()
        lo_all = (cpos + 1) * CH <= SEQ - 50
        hi_all = cpos * CH >= SEQ - 50

        for h in range(2):
            def pos_step(t, carry, h=h):
                for c in range(2 * DM // LANES):
                    wrc[h, t, 4 + c // 4, pl.ds((c % 4) * LANES, LANES)] = (
                        gb45[cpos, h * CHH + t, pl.ds(c * LANES, LANES)])
                for c in range(DM // LANES):
                    if lo_all:
                        vec = w6lo[c]
                    elif hi_all:
                        vec = w6hi[c]
                    else:
                        vec = jnp.where(
                            cpos * CH + h * CHH + t >= SEQ - 50,
                            w6hi[c], w6lo[c])
                    wrc[h, t, 6, pl.ds(c * LANES, LANES)] = vec
                return carry
            lax.fori_loop(0, CHH, pos_step, 0)

        pltpu.async_copy(wrc.at[0],
                         out.at[pl.ds(tbase + cpos * CH, CHH)], wsem)
        pltpu.async_copy(wrc.at[1],
                         out.at[pl.ds(tbase + cpos * CH + CHH, CHH)], wsem)
        fire(jnp.int32(0), cpos, xidxA, gbA, semA)
        fire(jnp.int32(1), cpos, xidxB, gbB, semB)

        def seg_body(j, carry):
            i = 2 * j
            for gb, xi, sem, ii in ((gbA, xidxA, semA, i),
                                    (gbB, xidxB, semB, i + 1)):
                drain_g(gb, sem)
                drain_w()
                unpack_half(gb, 0)
                write(ii, cpos, 0)
                drain_w()
                unpack_half(gb, 1)
                write(ii, cpos, 1)
                fire(jnp.minimum(ii + 2, ROWS_PER_W - 1), cpos, xi, gb, sem)
            return carry
        lax.fori_loop(0, ROWS_PER_W // 2, seg_body, 0)

        drain_g(gbA, semA)
        drain_g(gbB, semB)

    drain_w()
    drain_w()


def _pair(w, side):
    z = jnp.zeros_like(w)
    cols = (w, z) if side == 0 else (z, w)
    return jnp.concatenate(cols, axis=1)


def kernel(x, W0, W1, W2, W3, W4, W5, W6):
    xTc = jnp.transpose(
        x.astype(jnp.int32).reshape(BATCH, NCHUNK, CH, NV),
        (0, 1, 3, 2)).reshape(-1)
    Wd = [jnp.concatenate([w, w], axis=1) for w in (W0, W1, W2, W3)]
    WL4, WR5 = _pair(W4, 0), _pair(W5, 1)
    W6f = W6.reshape(2 * DM)

    mesh = plsc.VectorSubcoreMesh(core_axis_name="c", subcore_axis_name="s")
    f = pl.kernel(
        _body,
        out_type=jax.ShapeDtypeStruct((TOK, 7, DM), jnp.float32),
        mesh=mesh,
        scratch_types=[
            pltpu.VMEM((NV * CH,), jnp.int32),
            pltpu.VMEM((NV * CH,), jnp.int32),
            pltpu.VMEM((IPAD,), jnp.int32),
            pltpu.VMEM((IPAD,), jnp.int32),
            pltpu.VMEM((2 * DM,), jnp.float32),
            pltpu.VMEM((2, CHH, 7, DM), jnp.float32),
            pltpu.VMEM((NV, CH, 2 * DM), jnp.float32),
            pltpu.VMEM((NV, CH, 2 * DM), jnp.float32),
            pltpu.VMEM((NCHUNK, CH, 2 * DM), jnp.float32),
            pltpu.SemaphoreType.DMA,
            pltpu.SemaphoreType.DMA,
            pltpu.SemaphoreType.DMA,
        ],
    )
    out = f(xTc, *Wd, WL4, WR5, W6f)
    return out.reshape(BATCH, SEQ, 7, DM)


# speedup vs baseline: 1.3219x; 1.0142x over previous
import jax
import jax.numpy as jnp
from jax import lax
from jax.experimental import pallas as pl
from jax.experimental.pallas import tpu as pltpu
from jax.experimental.pallas import tpu_sc as plsc

SEQ = 200
BATCH = 1024
DM = 64
NV = 4
LANES = 16
NC, NS = 2, 16
NW = NC * NS
ROWS_PER_W = BATCH // NW
CH = 40
NCHUNK = SEQ // CH
WTOK = ROWS_PER_W * SEQ
NPAD = NCHUNK * CH + 0
IPAD = 208
TOK = BATCH * SEQ
CHH = CH // 4
NQ = CH // CHH


def _body(xTc, Wd0, Wd1, Wd2, Wd3, WL4, WR5, W6f, out,
          xidxA, xidxB, pidx_s, pidx_f, w6v, wrc, gbA, gbB, gb45,
          semA, semB, wsem):
    wid = lax.axis_index("s") * NC + lax.axis_index("c")
    tbase = wid * WTOK
    gbase = wid * ROWS_PER_W * NCHUNK
    tables = (Wd0, Wd1, Wd2, Wd3)

    iota = lax.iota(jnp.int32, LANES)
    for i in range(IPAD // LANES):
        s = iota + (i * LANES)
        pidx_s[pl.ds(i * LANES, LANES)] = jnp.minimum(s, SEQ - 1)
        pidx_f[pl.ds(i * LANES, LANES)] = jnp.clip(s - 149, 0, 50)

    pltpu.sync_copy(W6f, w6v)
    w6lo = [w6v[pl.ds(c * LANES, LANES)] for c in range(DM // LANES)]
    w6hi = [w6v[pl.ds(DM + c * LANES, LANES)] for c in range(DM // LANES)]
    for c in range(NCHUNK):
        dst = gb45.at[c]
        pltpu.async_copy(WL4.at[pidx_s.at[pl.ds(c * CH, CH)]],
                         dst, semA).wait()
        pltpu.async_copy(WR5.at[pidx_f.at[pl.ds(c * CH, CH)]],
                         dst, semA, add=True).wait()

    def fire(i, cpos, xi, gb, sem):
        gid = gbase + i * NCHUNK + cpos
        pltpu.sync_copy(xTc.at[pl.ds(gid * (NV * CH), NV * CH)], xi)
        for v in range(NV):
            pltpu.async_copy(tables[v].at[xi.at[pl.ds(v * CH, CH)]],
                             gb.at[v], sem)

    def drain_g(gb, sem):
        for v in range(NV):
            pltpu.make_async_copy(Wd0.at[pl.ds(0, CH)], gb.at[v], sem).wait()

    def drain_w():
        pltpu.make_async_copy(wrc.at[0], out.at[pl.ds(0, CHH)], wsem).wait()

    def unpack_half(gb, h):
        def step(t, carry):
            for v in range(NV):
                for c in range(DM // LANES):
                    wrc[h, t, v, pl.ds(c * LANES, LANES)] = (
                        gb[v, h * CHH + t, pl.ds(c * LANES, LANES)])
            return carry
        lax.fori_loop(0, CHH, step, 0)

    def write(i, cpos, h):
        pltpu.async_copy(
            wrc.at[h],
            out.at[pl.ds(tbase + i * SEQ + cpos * CH + h * CHH, CHH)], wsem)

    for h in range(NQ):
        pltpu.async_copy(wrc.at[h], out.at[pl.ds(tbase + h * CHH, CHH)],
                         wsem)

    for cpos in range(NCHUNK):
        for _ in range(NQ):
            drain_w()
        lo_all = (cpos + 1) * CH <= SEQ - 50
        hi_all = cpos * CH >= SEQ - 50

        for h in range(NQ):
            def pos_step(t, carry, h=h):
                for c in range(2 * DM // LANES):
                    wrc[h, t, 4 + c // 4, pl.ds((c % 4) * LANES, LANES)] = (
                        gb45[cpos, h * CHH + t, pl.ds(c * LANES, LANES)])
                for c in range(DM // LANES):
                    if lo_all:
                        vec = w6lo[c]
                    elif hi_all:
                        vec = w6hi[c]
                    else:
                        vec = jnp.where(
                            cpos * CH + h * CHH + t >= SEQ - 50,
                            w6hi[c], w6lo[c])
                    wrc[h, t, 6, pl.ds(c * LANES, LANES)] = vec
                return carry
            lax.fori_loop(0, CHH, pos_step, 0)

        for h in range(NQ):
            pltpu.async_copy(
                wrc.at[h],
                out.at[pl.ds(tbase + cpos * CH + h * CHH, CHH)], wsem)
        fire(jnp.int32(0), cpos, xidxA, gbA, semA)
        fire(jnp.int32(1), cpos, xidxB, gbB, semB)

        def seg_body(j, carry):
            i = 2 * j
            for gb, xi, sem, ii in ((gbA, xidxA, semA, i),
                                    (gbB, xidxB, semB, i + 1)):
                drain_g(gb, sem)
                for h in range(NQ):
                    drain_w()
                    unpack_half(gb, h)
                    write(ii, cpos, h)
                fire(jnp.minimum(ii + 2, ROWS_PER_W - 1), cpos, xi, gb, sem)
            return carry
        lax.fori_loop(0, ROWS_PER_W // 2, seg_body, 0)

        drain_g(gbA, semA)
        drain_g(gbB, semB)

    for _ in range(NQ):
        drain_w()


def _pair(w, side):
    z = jnp.zeros_like(w)
    cols = (w, z) if side == 0 else (z, w)
    return jnp.concatenate(cols, axis=1)


def kernel(x, W0, W1, W2, W3, W4, W5, W6):
    xTc = jnp.transpose(
        x.astype(jnp.int32).reshape(BATCH, NCHUNK, CH, NV),
        (0, 1, 3, 2)).reshape(-1)
    Wd = [jnp.concatenate([w, w], axis=1) for w in (W0, W1, W2, W3)]
    WL4, WR5 = _pair(W4, 0), _pair(W5, 1)
    W6f = W6.reshape(2 * DM)

    mesh = plsc.VectorSubcoreMesh(core_axis_name="c", subcore_axis_name="s")
    f = pl.kernel(
        _body,
        out_type=jax.ShapeDtypeStruct((TOK, 7, DM), jnp.float32),
        mesh=mesh,
        scratch_types=[
            pltpu.VMEM((NV * CH,), jnp.int32),
            pltpu.VMEM((NV * CH,), jnp.int32),
            pltpu.VMEM((IPAD,), jnp.int32),
            pltpu.VMEM((IPAD,), jnp.int32),
            pltpu.VMEM((2 * DM,), jnp.float32),
            pltpu.VMEM((NQ, CHH, 7, DM), jnp.float32),
            pltpu.VMEM((NV, CH, 2 * DM), jnp.float32),
            pltpu.VMEM((NV, CH, 2 * DM), jnp.float32),
            pltpu.VMEM((NCHUNK, CH, 2 * DM), jnp.float32),
            pltpu.SemaphoreType.DMA,
            pltpu.SemaphoreType.DMA,
            pltpu.SemaphoreType.DMA,
        ],
    )
    out = f(xTc, *Wd, WL4, WR5, W6f)
    return out.reshape(BATCH, SEQ, 7, DM)
